# initial kernel scaffold (unmeasured)
import jax
import jax.numpy as jnp
from jax import lax
from jax.experimental import pallas as pl
from jax.experimental.pallas import tpu as pltpu

N_DEV = 4


def kernel(x, router_W, route_idx, expert_W, shared_W):
    n_tok, d = x.shape
    e_local = expert_W.shape[0]
    h = shared_W.shape[1]
    n_experts = router_W.shape[1]

    def body(x_ref, rw_ref, idx_ref, ew_ref, sw_ref, out_ref,
             comm_ref, send_sems, recv_sems):
        my = lax.axis_index("i")
        left = lax.rem(my + N_DEV - 1, N_DEV)
        right = lax.rem(my + 1, N_DEV)

        barrier_sem = pltpu.get_barrier_semaphore()
        for nbr in (left, right):
            pl.semaphore_signal(
                barrier_sem, inc=1,
                device_id=(nbr,), device_id_type=pl.DeviceIdType.MESH,
            )
        pl.semaphore_wait(barrier_sem, 2)

        x32 = x_ref[:]
        scores = jnp.dot(x32, rw_ref[:], preferred_element_type=jnp.float32)
        smax = jnp.max(scores, axis=-1, keepdims=True)
        pexp = jnp.exp(scores - smax)
        probs = pexp / jnp.sum(pexp, axis=-1, keepdims=True)
        ridx = idx_ref[:]
        eids = lax.broadcasted_iota(jnp.int32, (n_tok, n_experts), 1)
        p = jnp.sum(jnp.where(eids == ridx, probs, 0.0), axis=1,
                    keepdims=True)

        xb = x32.astype(jnp.bfloat16)
        xp = (x32 * p).astype(jnp.bfloat16)

        for e in range(e_local):
            comm_ref[0, e, :, :] = ew_ref[e, :, :].astype(jnp.bfloat16)

        def contrib(acc, slot, base):
            w = comm_ref[slot]
            for e in range(e_local):
                sel = ridx == (base + e)
                xe = jnp.where(sel, xp, jnp.bfloat16(0.0))
                acc = acc + jnp.dot(w[e], preferred_element_type=jnp.float32,
                                    a=None) if False else acc + jnp.dot(
                    xe, w[e], preferred_element_type=jnp.float32)
            return acc

        rdma0 = pltpu.make_async_remote_copy(
            src_ref=comm_ref.at[0], dst_ref=comm_ref.at[1],
            send_sem=send_sems.at[0], recv_sem=recv_sems.at[0],
            device_id=(right,), device_id_type=pl.DeviceIdType.MESH,
        )
        rdma0.start()

        acc = jnp.dot(xb, sw_ref[:].astype(jnp.bfloat16),
                      preferred_element_type=jnp.float32)
        acc = contrib(acc, 0, my * e_local)

        rdma0.wait()

        for hh in range(1, N_DEV - 1):
            rdma = pltpu.make_async_remote_copy(
                src_ref=comm_ref.at[hh], dst_ref=comm_ref.at[hh + 1],
                send_sem=send_sems.at[hh], recv_sem=recv_sems.at[hh],
                device_id=(right,), device_id_type=pl.DeviceIdType.MESH,
            )
            rdma.start()
            origin = lax.rem(my - hh + N_DEV, N_DEV)
            acc = contrib(acc, hh, origin * e_local)
            rdma.wait()

        origin = lax.rem(my + 1, N_DEV)
        acc = contrib(acc, N_DEV - 1, origin * e_local)

        out_ref[:] = acc

    return pl.pallas_call(
        body,
        out_shape=jax.ShapeDtypeStruct((n_tok, h), jnp.float32),
        in_specs=[pl.BlockSpec(memory_space=pltpu.VMEM)] * 5,
        out_specs=pl.BlockSpec(memory_space=pltpu.VMEM),
        scratch_shapes=[
            pltpu.VMEM((N_DEV, e_local, d, h), jnp.bfloat16),
            pltpu.SemaphoreType.DMA((N_DEV - 1,)),
            pltpu.SemaphoreType.DMA((N_DEV - 1,)),
        ],
        compiler_params=pltpu.CompilerParams(collective_id=0),
    )(x, router_W, route_idx, expert_W, shared_W)


# baseline (device time: 332158 ns/iter reference)
import jax
import jax.numpy as jnp
from jax import lax
from jax.experimental import pallas as pl
from jax.experimental.pallas import tpu as pltpu

N_DEV = 4
TILE = 256


def kernel(x, router_W, route_idx, expert_W, shared_W):
    n_tok, d = x.shape
    e_local = expert_W.shape[0]
    h = shared_W.shape[1]
    n_experts = router_W.shape[1]
    n_tiles = n_tok // TILE

    def body(x_ref, rw_ref, idx_ref, ew_ref, sw_ref, out_ref,
             comm_ref, xp_ref, stage_ref, copy_sem, send_sems, recv_sems):
        my = lax.axis_index("i")
        left = lax.rem(my + N_DEV - 1, N_DEV)
        right = lax.rem(my + 1, N_DEV)

        barrier_sem = pltpu.get_barrier_semaphore()
        for nbr in (left, right):
            pl.semaphore_signal(
                barrier_sem, inc=1,
                device_id=(nbr,), device_id_type=pl.DeviceIdType.MESH,
            )
        pl.semaphore_wait(barrier_sem, 2)

        rw = rw_ref[:]
        def route_body(t, carry):
            rows = pl.ds(t * TILE, TILE)
            xt = x_ref[rows, :]
            scores = jnp.dot(xt, rw, preferred_element_type=jnp.float32)
            smax = jnp.max(scores, axis=-1, keepdims=True)
            pexp = jnp.exp(scores - smax)
            probs = pexp / jnp.sum(pexp, axis=-1, keepdims=True)
            rid = idx_ref[rows, :]
            eids = lax.broadcasted_iota(jnp.int32, (TILE, n_experts), 1)
            pt = jnp.sum(jnp.where(eids == rid, probs, 0.0), axis=1,
                         keepdims=True)
            xp_ref[rows, :] = (xt * pt).astype(jnp.bfloat16)
            return carry
        lax.fori_loop(0, n_tiles, route_body, 0)

        for e in range(e_local):
            cp = pltpu.make_async_copy(ew_ref.at[e], stage_ref, copy_sem)
            cp.start()
            cp.wait()
            comm_ref[0, e, :, :] = stage_ref[:].astype(jnp.bfloat16)

        def hop_compute(slot, base):
            def tile_body(t, carry):
                rows = pl.ds(t * TILE, TILE)
                xpt = xp_ref[rows, :]
                rid = idx_ref[rows, :]
                acc = out_ref[rows, :]
                for e in range(e_local):
                    sel = rid == (base + e)
                    xe = jnp.where(sel, xpt, jnp.bfloat16(0.0))
                    acc = acc + jnp.dot(xe, comm_ref[slot, e],
                                        preferred_element_type=jnp.float32)
                out_ref[rows, :] = acc
                return carry
            lax.fori_loop(0, n_tiles, tile_body, 0)

        rdma0 = pltpu.make_async_remote_copy(
            src_ref=comm_ref.at[0], dst_ref=comm_ref.at[1],
            send_sem=send_sems.at[0], recv_sem=recv_sems.at[0],
            device_id=(right,), device_id_type=pl.DeviceIdType.MESH,
        )
        rdma0.start()

        swb = sw_ref[:].astype(jnp.bfloat16)
        def shared_body(t, carry):
            rows = pl.ds(t * TILE, TILE)
            xbt = x_ref[rows, :].astype(jnp.bfloat16)
            out_ref[rows, :] = jnp.dot(xbt, swb,
                                       preferred_element_type=jnp.float32)
            return carry
        lax.fori_loop(0, n_tiles, shared_body, 0)

        hop_compute(0, my * e_local)
        rdma0.wait()

        for hh in range(1, N_DEV - 1):
            rdma = pltpu.make_async_remote_copy(
                src_ref=comm_ref.at[hh], dst_ref=comm_ref.at[hh + 1],
                send_sem=send_sems.at[hh], recv_sem=recv_sems.at[hh],
                device_id=(right,), device_id_type=pl.DeviceIdType.MESH,
            )
            rdma.start()
            origin = lax.rem(my - hh + 2 * N_DEV, N_DEV)
            hop_compute(hh, origin * e_local)
            rdma.wait()

        hop_compute(N_DEV - 1, right * e_local)

    return pl.pallas_call(
        body,
        out_shape=jax.ShapeDtypeStruct((n_tok, h), jnp.float32),
        in_specs=[
            pl.BlockSpec(memory_space=pltpu.VMEM),
            pl.BlockSpec(memory_space=pltpu.VMEM),
            pl.BlockSpec(memory_space=pltpu.VMEM),
            pl.BlockSpec(memory_space=pl.ANY),
            pl.BlockSpec(memory_space=pltpu.VMEM),
        ],
        out_specs=pl.BlockSpec(memory_space=pltpu.VMEM),
        scratch_shapes=[
            pltpu.VMEM((N_DEV, e_local, d, h), jnp.bfloat16),
            pltpu.VMEM((n_tok, d), jnp.bfloat16),
            pltpu.VMEM((d, h), jnp.float32),
            pltpu.SemaphoreType.DMA,
            pltpu.SemaphoreType.DMA((N_DEV - 1,)),
            pltpu.SemaphoreType.DMA((N_DEV - 1,)),
        ],
        compiler_params=pltpu.CompilerParams(
            collective_id=0, vmem_limit_bytes=58 * 1024 * 1024,
        ),
    )(x, router_W, route_idx, expert_W, shared_W)


# device time: 189849 ns/iter; 1.7496x vs baseline; 1.7496x over previous
import jax
import jax.numpy as jnp
from jax import lax
from jax.experimental import pallas as pl
from jax.experimental.pallas import tpu as pltpu

N_DEV = 4
TILE = 256


def kernel(x, router_W, route_idx, expert_W, shared_W):
    n_tok, d = x.shape
    e_local = expert_W.shape[0]
    h = shared_W.shape[1]
    n_experts = router_W.shape[1]
    n_tiles = n_tok // TILE
    half = e_local // 2

    ew_bf = expert_W.astype(jnp.bfloat16)

    def body(x_ref, rw_ref, idx_ref, ew_ref, sw_ref, out_ref,
             comm_ref, xp_ref, send_sems, recv_sems):
        my = lax.axis_index("i")
        left = lax.rem(my + N_DEV - 1, N_DEV)
        right = lax.rem(my + 1, N_DEV)
        opp = lax.rem(my + 2, N_DEV)

        barrier_sem = pltpu.get_barrier_semaphore()
        for nbr in (left, right):
            pl.semaphore_signal(
                barrier_sem, inc=1,
                device_id=(nbr,), device_id_type=pl.DeviceIdType.MESH,
            )
        pl.semaphore_wait(barrier_sem, 2)

        send_own_r = pltpu.make_async_remote_copy(
            src_ref=ew_ref, dst_ref=comm_ref.at[pl.ds(0, e_local)],
            send_sem=send_sems.at[0], recv_sem=recv_sems.at[0],
            device_id=(right,), device_id_type=pl.DeviceIdType.MESH,
        )
        send_own_l = pltpu.make_async_remote_copy(
            src_ref=ew_ref, dst_ref=comm_ref.at[pl.ds(e_local, e_local)],
            send_sem=send_sems.at[1], recv_sem=recv_sems.at[1],
            device_id=(left,), device_id_type=pl.DeviceIdType.MESH,
        )
        fwd_r = pltpu.make_async_remote_copy(
            src_ref=comm_ref.at[pl.ds(0, half)],
            dst_ref=comm_ref.at[pl.ds(2 * e_local, half)],
            send_sem=send_sems.at[2], recv_sem=recv_sems.at[2],
            device_id=(right,), device_id_type=pl.DeviceIdType.MESH,
        )
        fwd_l = pltpu.make_async_remote_copy(
            src_ref=comm_ref.at[pl.ds(e_local + half, half)],
            dst_ref=comm_ref.at[pl.ds(2 * e_local + half, half)],
            send_sem=send_sems.at[3], recv_sem=recv_sems.at[3],
            device_id=(left,), device_id_type=pl.DeviceIdType.MESH,
        )

        send_own_r.start()
        send_own_l.start()

        rw = rw_ref[:]
        def route_body(t, carry):
            rows = pl.ds(t * TILE, TILE)
            xt = x_ref[rows, :]
            scores = jnp.dot(xt, rw, preferred_element_type=jnp.float32)
            smax = jnp.max(scores, axis=-1, keepdims=True)
            pexp = jnp.exp(scores - smax)
            probs = pexp / jnp.sum(pexp, axis=-1, keepdims=True)
            rid = idx_ref[rows, :]
            eids = lax.broadcasted_iota(jnp.int32, (TILE, n_experts), 1)
            pt = jnp.sum(jnp.where(eids == rid, probs, 0.0), axis=1,
                         keepdims=True)
            xp_ref[rows, :] = (xt * pt).astype(jnp.bfloat16)
            return carry
        lax.fori_loop(0, n_tiles, route_body, 0)

        swb = sw_ref[:].astype(jnp.bfloat16)
        def shared_body(t, carry):
            rows = pl.ds(t * TILE, TILE)
            xbt = x_ref[rows, :].astype(jnp.bfloat16)
            out_ref[rows, :] = jnp.dot(xbt, swb,
                                       preferred_element_type=jnp.float32)
            return carry
        lax.fori_loop(0, n_tiles, shared_body, 0)

        def contrib(w_ref, off, base):
            def tile_body(t, carry):
                rows = pl.ds(t * TILE, TILE)
                xpt = xp_ref[rows, :]
                rid = idx_ref[rows, :]
                acc = out_ref[rows, :]
                for e in range(e_local):
                    sel = rid == (base + e)
                    xe = jnp.where(sel, xpt, jnp.bfloat16(0.0))
                    acc = acc + jnp.dot(xe, w_ref[off + e],
                                        preferred_element_type=jnp.float32)
                out_ref[rows, :] = acc
                return carry
            lax.fori_loop(0, n_tiles, tile_body, 0)

        contrib(ew_ref, 0, my * e_local)

        send_own_r.wait_recv()
        fwd_r.start()
        send_own_l.wait_recv()
        fwd_l.start()

        contrib(comm_ref, 0, left * e_local)
        contrib(comm_ref, e_local, right * e_local)

        fwd_r.wait_recv()
        fwd_l.wait_recv()
        contrib(comm_ref, 2 * e_local, opp * e_local)

        send_own_r.wait_send()
        send_own_l.wait_send()
        fwd_r.wait_send()
        fwd_l.wait_send()

    return pl.pallas_call(
        body,
        out_shape=jax.ShapeDtypeStruct((n_tok, h), jnp.float32),
        in_specs=[
            pl.BlockSpec(memory_space=pltpu.VMEM),
            pl.BlockSpec(memory_space=pltpu.VMEM),
            pl.BlockSpec(memory_space=pltpu.VMEM),
            pl.BlockSpec(memory_space=pltpu.VMEM),
            pl.BlockSpec(memory_space=pltpu.VMEM),
        ],
        out_specs=pl.BlockSpec(memory_space=pltpu.VMEM),
        scratch_shapes=[
            pltpu.VMEM((3 * e_local, d, h), jnp.bfloat16),
            pltpu.VMEM((n_tok, d), jnp.bfloat16),
            pltpu.SemaphoreType.DMA((4,)),
            pltpu.SemaphoreType.DMA((4,)),
        ],
        compiler_params=pltpu.CompilerParams(
            collective_id=0, vmem_limit_bytes=58 * 1024 * 1024,
        ),
    )(x, router_W, route_idx, ew_bf, shared_W)
